# Initial kernel scaffold; baseline (speedup 1.0000x reference)
import jax
import jax.numpy as jnp
from jax.experimental import pallas as pl


def kernel(x, g_edge_index, lg_edge_index, W1, b1, Wc1, bc1, Wc2, bc2, W2, b2):
    raise NotImplementedError("write your pallas kernel here")


# speedup vs baseline: 2.5954x; 2.5954x over previous
import functools

import jax
import jax.numpy as jnp
import numpy as np
from jax.experimental import pallas as pl

N_NODES = 10000
D_IN = 128
E_G = 320000
E_LG = 5120000
HIDDEN = 52
F = 64


def _mm_bias_body(a_ref, w_ref, b_ref, o_ref):
    o_ref[...] = jnp.dot(a_ref[...], w_ref[...],
                         preferred_element_type=jnp.float32) + b_ref[...]


def _mm_bias(a, w, b, blk):
    n, k = a.shape
    m = w.shape[1]
    return pl.pallas_call(
        _mm_bias_body,
        grid=(n // blk,),
        in_specs=[
            pl.BlockSpec((blk, k), lambda i: (i, 0)),
            pl.BlockSpec((k, m), lambda i: (0, 0)),
            pl.BlockSpec((1, m), lambda i: (0, 0)),
        ],
        out_specs=pl.BlockSpec((blk, m), lambda i: (i, 0)),
        out_shape=jax.ShapeDtypeStruct((n, m), jnp.float32),
    )(a, w, b)


def _fused_layer_body(a_ref, u_ref, v_ref, b_ref, w_ref, c_ref, o_ref):
    h = jax.nn.relu(u_ref[...] * a_ref[...] + b_ref[...])
    o_ref[...] = jnp.dot(v_ref[...] * h, w_ref[...],
                         preferred_element_type=jnp.float32) + c_ref[...]


def _fused_layer(a, u, v, b, w, c, blk):
    n, k = a.shape
    m = w.shape[1]
    return pl.pallas_call(
        _fused_layer_body,
        grid=(n // blk,),
        in_specs=[
            pl.BlockSpec((blk, k), lambda i: (i, 0)),
            pl.BlockSpec((blk, 1), lambda i: (i, 0)),
            pl.BlockSpec((blk, 1), lambda i: (i, 0)),
            pl.BlockSpec((1, k), lambda i: (0, 0)),
            pl.BlockSpec((k, m), lambda i: (0, 0)),
            pl.BlockSpec((1, m), lambda i: (0, 0)),
        ],
        out_specs=pl.BlockSpec((blk, m), lambda i: (i, 0)),
        out_shape=jax.ShapeDtypeStruct((n, m), jnp.float32),
    )(a, u, v, b, w, c)


def _dis_body(deg_ref, o_ref):
    o_ref[...] = jax.lax.rsqrt(deg_ref[...] + 1.0)


def _dis_from_deg(deg, blk):
    n = deg.shape[0]
    return pl.pallas_call(
        _dis_body,
        grid=(n // blk,),
        in_specs=[pl.BlockSpec((blk, 1), lambda i: (i, 0))],
        out_specs=pl.BlockSpec((blk, 1), lambda i: (i, 0)),
        out_shape=jax.ShapeDtypeStruct((n, 1), jnp.float32),
    )(deg)


def kernel(x, g_edge_index, lg_edge_index, W1, b1, Wc1, bc1, Wc2, bc2, W2, b2):
    f32 = jnp.float32
    Wp = jnp.zeros((D_IN, 2 * F), f32)
    Wp = Wp.at[:, :HIDDEN].set(W1[:D_IN])
    Wp = Wp.at[:, F:F + HIDDEN].set(W1[D_IN:])
    bp = jnp.zeros((1, 2 * F), f32).at[0, F:F + HIDDEN].set(b1)
    Wc1p = jnp.zeros((F, F), f32).at[:HIDDEN, :HIDDEN].set(Wc1)
    bc1p = jnp.zeros((1, F), f32).at[0, :HIDDEN].set(bc1)
    Wc2p = jnp.zeros((F, F), f32).at[:HIDDEN, :HIDDEN].set(Wc2)
    bc2p = jnp.zeros((1, F), f32).at[0, :HIDDEN].set(bc2)
    W2p = jnp.zeros((F, 1), f32).at[:HIDDEN, :].set(W2)
    b2p = b2.reshape(1, 1)
    ones_n = jnp.ones((E_G, 1), f32)
    zeros_f = jnp.zeros((1, F), f32)
    zeros_1 = jnp.zeros((1, 1), f32)

    Zx = _mm_bias(x, Wp, bp, blk=2000)
    xa = Zx[:, :F]
    xb = Zx[:, F:]

    gsrc, gdst = g_edge_index[0], g_edge_index[1]
    lsrc, ldst = lg_edge_index[0], lg_edge_index[1]

    h0 = jax.nn.relu(jnp.take(xa, gsrc, axis=0) + jnp.take(xb, gdst, axis=0))

    deg = jnp.zeros((E_G, 1), f32).at[ldst, 0].add(1.0)
    dis = _dis_from_deg(deg, blk=4000)

    P1 = _fused_layer(h0, ones_n, dis, zeros_f, Wc1p, zeros_f, blk=4000)
    acc1 = P1.at[ldst].add(jnp.take(P1, lsrc, axis=0))
    P2 = _fused_layer(acc1, dis, dis, bc1p, Wc2p, zeros_f, blk=4000)
    acc2 = P2.at[ldst].add(jnp.take(P2, lsrc, axis=0))
    out = _fused_layer(acc2, dis, ones_n, bc2p, W2p, b2p, blk=4000)
    return out


# speedup vs baseline: 9.2857x; 3.5777x over previous
import functools

import jax
import jax.numpy as jnp
import numpy as np
from jax import lax
from jax.experimental import pallas as pl
from jax.experimental.pallas import tpu as pltpu
from jax.experimental.pallas import tpu_sc as plsc

N_NODES = 10000
D_IN = 128
E_G = 320000
E_LG = 5120000
HIDDEN = 52
F = 64

NC, NS = 2, 16
NW = NC * NS
_SC_MESH = dict(core_axis_name="c", subcore_axis_name="s")


def _wid():
    return lax.axis_index("s") * NC + lax.axis_index("c")


_H0_W = 128
_H0_PER = E_G // NW


def _h0_window(xa_hbm, xb_hbm, src_hbm, dst_hbm, out_hbm, isrc, idst, ra, rb,
               s1, s2, off, n):
    pltpu.sync_copy(src_hbm.at[pl.ds(off, n)], isrc.at[pl.ds(0, n)])
    pltpu.sync_copy(dst_hbm.at[pl.ds(off, n)], idst.at[pl.ds(0, n)])
    ca = pltpu.async_copy(xa_hbm.at[isrc.at[pl.ds(0, n)]], ra.at[pl.ds(0, n)], s1)
    cb = pltpu.async_copy(xb_hbm.at[idst.at[pl.ds(0, n)]], rb.at[pl.ds(0, n)], s2)
    ca.wait()
    cb.wait()

    def row(r, _):
        for j in range(F // 16):
            sl = pl.ds(j * 16, 16)
            ra[r, sl] = jnp.maximum(ra[r, sl] + rb[r, sl], 0.0)
        return 0

    lax.fori_loop(0, n, row, 0, unroll=4)
    pltpu.sync_copy(ra.at[pl.ds(0, n)], out_hbm.at[pl.ds(off, n)])


def _h0_body(xa_hbm, xb_hbm, src_hbm, dst_hbm, out_hbm, isrc, idst, ra, rb,
             s1, s2):
    base = _wid() * _H0_PER
    nwin = _H0_PER // _H0_W

    def win(w, _):
        _h0_window(xa_hbm, xb_hbm, src_hbm, dst_hbm, out_hbm, isrc, idst,
                   ra, rb, s1, s2, base + w * _H0_W, _H0_W)
        return 0

    lax.fori_loop(0, nwin, win, 0)
    tail = _H0_PER - nwin * _H0_W
    if tail:
        _h0_window(xa_hbm, xb_hbm, src_hbm, dst_hbm, out_hbm, isrc, idst,
                   ra, rb, s1, s2, base + nwin * _H0_W, tail)


def _sc_h0(xa, xb, gsrc, gdst):
    return pl.kernel(
        _h0_body,
        out_type=jax.ShapeDtypeStruct((E_G, F), jnp.float32),
        mesh=plsc.VectorSubcoreMesh(**_SC_MESH),
        compiler_params=pltpu.CompilerParams(use_tc_tiling_on_sc=False),
        scratch_types=[
            pltpu.VMEM((_H0_W,), jnp.int32),
            pltpu.VMEM((_H0_W,), jnp.int32),
            pltpu.VMEM((_H0_W, F), jnp.float32),
            pltpu.VMEM((_H0_W, F), jnp.float32),
            pltpu.SemaphoreType.DMA,
            pltpu.SemaphoreType.DMA,
        ],
    )(xa, xb, gsrc, gdst)


_DEG_CHUNK = 3200
_DEG_TEDGES = E_LG // NC // NS
_DEG_SLICE = E_G // NS


def _copy128(dst, src, off):
    for j in range(8):
        dst[pl.ds(j * 16, 16)] = src[pl.ds(off + j * 16, 16)]


def _deg_body(ldst_hbm, out_hbm, ibig, idxw, ones, zbuf, deg_sh, s1):
    cid = lax.axis_index("c")
    sid = lax.axis_index("s")

    def fill(j, _):
        ones[pl.ds(j * 16, 16)] = jnp.full((16,), 1.0, jnp.float32)
        return 0

    lax.fori_loop(0, 128 // 16, fill, 0)

    def zero(i, _):
        zbuf[pl.ds(i * 16, 16)] = jnp.zeros((16,), jnp.float32)
        return 0

    lax.fori_loop(0, _DEG_SLICE // 16, zero, 0, unroll=8)
    pltpu.sync_copy(zbuf, deg_sh.at[pl.ds(sid * _DEG_SLICE, _DEG_SLICE)])
    plsc.subcore_barrier()

    base = (cid * NS + sid) * _DEG_TEDGES

    def chunk(k, _):
        pltpu.sync_copy(ldst_hbm.at[pl.ds(base + k * _DEG_CHUNK, _DEG_CHUNK)],
                        ibig)

        def win(w, _):
            _copy128(idxw, ibig, w * 128)
            pltpu.sync_copy(ones, deg_sh.at[idxw], add=True)
            return 0

        lax.fori_loop(0, _DEG_CHUNK // 128, win, 0)
        return 0

    lax.fori_loop(0, _DEG_TEDGES // _DEG_CHUNK, chunk, 0)
    plsc.subcore_barrier()
    off = sid * _DEG_SLICE
    pltpu.sync_copy(deg_sh.at[pl.ds(off, _DEG_SLICE)], zbuf)
    pltpu.sync_copy(zbuf, out_hbm.at[cid, pl.ds(off, _DEG_SLICE)])


def _sc_deg(ldst):
    return pl.kernel(
        _deg_body,
        out_type=jax.ShapeDtypeStruct((NC, E_G), jnp.float32),
        mesh=plsc.VectorSubcoreMesh(**_SC_MESH),
        compiler_params=pltpu.CompilerParams(use_tc_tiling_on_sc=False),
        scratch_types=[
            pltpu.VMEM((_DEG_CHUNK,), jnp.int32),
            pltpu.VMEM((128,), jnp.int32),
            pltpu.VMEM((128,), jnp.float32),
            pltpu.VMEM((_DEG_SLICE,), jnp.float32),
            pltpu.VMEM_SHARED((E_G,), jnp.float32),
            pltpu.SemaphoreType.DMA,
        ],
    )(ldst)


_ACC_NB = 16
_ACC_R = E_G // _ACC_NB
_ACC_DUMP = _ACC_R
_ACC_S = 6400
_ACC_NWIN = _ACC_S // 128 + 1
_ACC_STRIP = E_LG // NS
_ACC_WSLICE = _ACC_R // NS


def _acc_body(p_hbm, lsrc_hbm, ldst_hbm, out_hbm,
              raw_s, raw_d, cs_flat, cd_flat, win_s, win_d, rows,
              acc_sh, s1):
    cid = lax.axis_index("c")
    sid = lax.axis_index("s")
    iota = lax.broadcasted_iota(jnp.int32, (16,), 0)

    def window(d):
        _copy128(win_s, cs_flat, d)
        _copy128(win_d, cd_flat, d)
        pltpu.async_copy(p_hbm.at[win_s], rows, s1).wait()
        pltpu.sync_copy(rows, acc_sh.at[win_d], add=True)

    for r_i in range(_ACC_NB // NC):
        rng = cid * (_ACC_NB // NC) + r_i
        lo = rng * _ACC_R
        off = sid * _ACC_WSLICE
        pltpu.sync_copy(p_hbm.at[pl.ds(lo + off, _ACC_WSLICE)],
                        acc_sh.at[pl.ds(off, _ACC_WSLICE)])
        plsc.subcore_barrier()

        def subchunk(k, ptr):
            pltpu.sync_copy(
                lsrc_hbm.at[pl.ds(sid * _ACC_STRIP + k * _ACC_S, _ACC_S)],
                raw_s)
            pltpu.sync_copy(
                ldst_hbm.at[pl.ds(sid * _ACC_STRIP + k * _ACC_S, _ACC_S)],
                raw_d)

            def scan(i, ptr):
                vs = raw_s[pl.ds(i * 16, 16)]
                vd = raw_d[pl.ds(i * 16, 16)]
                m = (vd >= lo) & (vd < lo + _ACC_R)
                csum = plsc.cumsum(m.astype(jnp.int32))
                pos = ptr + csum - 1
                plsc.store_scatter(cs_flat, [pos], vs, mask=m)
                plsc.store_scatter(cd_flat, [pos], vd - lo, mask=m)
                cnt = plsc.all_reduce_population_count(m)
                return ptr + cnt[0]

            ptr = lax.fori_loop(0, _ACC_S // 16, scan, ptr)

            def drain(d, _):
                @pl.when(d * 128 + 128 <= ptr)
                def _w():
                    window(d * 128)
                return 0

            lax.fori_loop(0, _ACC_NWIN, drain, 0)

            done = (ptr // 128) * 128
            for j in range(8):
                sl = pl.ds(j * 16, 16)
                cs_flat[sl] = cs_flat[pl.ds(done + j * 16, 16)]
                cd_flat[sl] = cd_flat[pl.ds(done + j * 16, 16)]
            return ptr - done

        ptr = lax.fori_loop(0, _ACC_STRIP // _ACC_S, subchunk, 0)

        @pl.when(ptr > 0)
        def _flush():
            for j in range(8):
                lane = j * 16 + iota
                mf = lane >= ptr
                plsc.store_scatter(cs_flat, [lane],
                                   jnp.zeros((16,), jnp.int32), mask=mf)
                plsc.store_scatter(cd_flat, [lane],
                                   jnp.full((16,), _ACC_DUMP, jnp.int32),
                                   mask=mf)
            window(0)

        plsc.subcore_barrier()
        pltpu.sync_copy(acc_sh.at[pl.ds(off, _ACC_WSLICE)],
                        out_hbm.at[pl.ds(lo + off, _ACC_WSLICE)])


def _sc_acc(p, lsrc, ldst):
    return pl.kernel(
        _acc_body,
        out_type=jax.ShapeDtypeStruct((E_G, F), jnp.float32),
        mesh=plsc.VectorSubcoreMesh(**_SC_MESH),
        compiler_params=pltpu.CompilerParams(use_tc_tiling_on_sc=False,
                                             needs_layout_passes=False),
        scratch_types=[
            pltpu.VMEM((_ACC_S,), jnp.int32),
            pltpu.VMEM((_ACC_S,), jnp.int32),
            pltpu.VMEM((8192,), jnp.int32),
            pltpu.VMEM((8192,), jnp.int32),
            pltpu.VMEM((128,), jnp.int32),
            pltpu.VMEM((128,), jnp.int32),
            pltpu.VMEM((128, F), jnp.float32),
            pltpu.VMEM_SHARED((_ACC_R + 8, F), jnp.float32),
            pltpu.SemaphoreType.DMA,
        ],
    )(p, lsrc, ldst)


def _mm_bias_body(a_ref, w_ref, b_ref, o_ref):
    o_ref[...] = jnp.dot(a_ref[...], w_ref[...],
                         preferred_element_type=jnp.float32) + b_ref[...]


def _mm_bias(a, w, b, blk):
    n, k = a.shape
    m = w.shape[1]
    return pl.pallas_call(
        _mm_bias_body,
        grid=(n // blk,),
        in_specs=[
            pl.BlockSpec((blk, k), lambda i: (i, 0)),
            pl.BlockSpec((k, m), lambda i: (0, 0)),
            pl.BlockSpec((1, m), lambda i: (0, 0)),
        ],
        out_specs=pl.BlockSpec((blk, m), lambda i: (i, 0)),
        out_shape=jax.ShapeDtypeStruct((n, m), jnp.float32),
    )(a, w, b)


def _fused_layer_body(a_ref, u_ref, v_ref, b_ref, w_ref, c_ref, o_ref):
    h = jax.nn.relu(u_ref[...] * a_ref[...] + b_ref[...])
    o_ref[...] = jnp.dot(v_ref[...] * h, w_ref[...],
                         preferred_element_type=jnp.float32) + c_ref[...]


def _fused_layer(a, u, v, b, w, c, blk):
    n, k = a.shape
    m = w.shape[1]
    return pl.pallas_call(
        _fused_layer_body,
        grid=(n // blk,),
        in_specs=[
            pl.BlockSpec((blk, k), lambda i: (i, 0)),
            pl.BlockSpec((blk, 1), lambda i: (i, 0)),
            pl.BlockSpec((blk, 1), lambda i: (i, 0)),
            pl.BlockSpec((1, k), lambda i: (0, 0)),
            pl.BlockSpec((k, m), lambda i: (0, 0)),
            pl.BlockSpec((1, m), lambda i: (0, 0)),
        ],
        out_specs=pl.BlockSpec((blk, m), lambda i: (i, 0)),
        out_shape=jax.ShapeDtypeStruct((n, m), jnp.float32),
    )(a, u, v, b, w, c)


def _dis_body(deg_ref, o_ref):
    o_ref[...] = jax.lax.rsqrt(deg_ref[0:1, :] + deg_ref[1:2, :] + 1.0)


def _dis_from_deg(degp, blk):
    n = degp.shape[1]
    return pl.pallas_call(
        _dis_body,
        out_shape=jax.ShapeDtypeStruct((1, n), jnp.float32),
    )(degp)


def kernel(x, g_edge_index, lg_edge_index, W1, b1, Wc1, bc1, Wc2, bc2, W2, b2):
    f32 = jnp.float32
    Wp = jnp.zeros((D_IN, 2 * F), f32)
    Wp = Wp.at[:, :HIDDEN].set(W1[:D_IN])
    Wp = Wp.at[:, F:F + HIDDEN].set(W1[D_IN:])
    bp = jnp.zeros((1, 2 * F), f32).at[0, F:F + HIDDEN].set(b1)
    Wc1p = jnp.zeros((F, F), f32).at[:HIDDEN, :HIDDEN].set(Wc1)
    bc1p = jnp.zeros((1, F), f32).at[0, :HIDDEN].set(bc1)
    Wc2p = jnp.zeros((F, F), f32).at[:HIDDEN, :HIDDEN].set(Wc2)
    bc2p = jnp.zeros((1, F), f32).at[0, :HIDDEN].set(bc2)
    W2p = jnp.zeros((F, 1), f32).at[:HIDDEN, :].set(W2)
    b2p = b2.reshape(1, 1)
    ones_n = jnp.ones((E_G, 1), f32)
    zeros_f = jnp.zeros((1, F), f32)
    zeros_1 = jnp.zeros((1, 1), f32)

    Zx = _mm_bias(x, Wp, bp, blk=2000)
    xa = Zx[:, :F]
    xb = Zx[:, F:]

    gsrc, gdst = g_edge_index[0], g_edge_index[1]
    lsrc, ldst = lg_edge_index[0], lg_edge_index[1]

    h0 = _sc_h0(xa, xb, gsrc, gdst)

    degp = _sc_deg(ldst)
    dis = _dis_from_deg(degp, blk=8000).reshape(E_G, 1)

    P1 = _fused_layer(h0, ones_n, dis, zeros_f, Wc1p, zeros_f, blk=4000)
    acc1 = _sc_acc(P1, lsrc, ldst)
    P2 = _fused_layer(acc1, dis, dis, bc1p, Wc2p, zeros_f, blk=4000)
    acc2 = _sc_acc(P2, lsrc, ldst)
    out = _fused_layer(acc2, dis, ones_n, bc2p, W2p, b2p, blk=4000)
    return out


# speedup vs baseline: 9.9231x; 1.0686x over previous
import functools

import jax
import jax.numpy as jnp
import numpy as np
from jax import lax
from jax.experimental import pallas as pl
from jax.experimental.pallas import tpu as pltpu
from jax.experimental.pallas import tpu_sc as plsc

N_NODES = 10000
D_IN = 128
E_G = 320000
E_LG = 5120000
HIDDEN = 52
F = 64

NC, NS = 2, 16
NW = NC * NS
_SC_MESH = dict(core_axis_name="c", subcore_axis_name="s")


def _wid():
    return lax.axis_index("s") * NC + lax.axis_index("c")


_H0_W = 128
_H0_PER = E_G // NW


def _h0_window(xa_hbm, xb_hbm, src_hbm, dst_hbm, out_hbm, isrc, idst, ra, rb,
               s1, s2, off, n):
    pltpu.sync_copy(src_hbm.at[pl.ds(off, n)], isrc.at[pl.ds(0, n)])
    pltpu.sync_copy(dst_hbm.at[pl.ds(off, n)], idst.at[pl.ds(0, n)])
    ca = pltpu.async_copy(xa_hbm.at[isrc.at[pl.ds(0, n)]], ra.at[pl.ds(0, n)], s1)
    cb = pltpu.async_copy(xb_hbm.at[idst.at[pl.ds(0, n)]], rb.at[pl.ds(0, n)], s2)
    ca.wait()
    cb.wait()

    def row(r, _):
        for j in range(F // 16):
            sl = pl.ds(j * 16, 16)
            ra[r, sl] = jnp.maximum(ra[r, sl] + rb[r, sl], 0.0)
        return 0

    lax.fori_loop(0, n, row, 0, unroll=4)
    pltpu.sync_copy(ra.at[pl.ds(0, n)], out_hbm.at[pl.ds(off, n)])


def _h0_body(xa_hbm, xb_hbm, src_hbm, dst_hbm, out_hbm, isrc, idst, ra, rb,
             s1, s2):
    base = _wid() * _H0_PER
    nwin = _H0_PER // _H0_W

    def win(w, _):
        _h0_window(xa_hbm, xb_hbm, src_hbm, dst_hbm, out_hbm, isrc, idst,
                   ra, rb, s1, s2, base + w * _H0_W, _H0_W)
        return 0

    lax.fori_loop(0, nwin, win, 0)
    tail = _H0_PER - nwin * _H0_W
    if tail:
        _h0_window(xa_hbm, xb_hbm, src_hbm, dst_hbm, out_hbm, isrc, idst,
                   ra, rb, s1, s2, base + nwin * _H0_W, tail)


def _sc_h0(xa, xb, gsrc, gdst):
    return pl.kernel(
        _h0_body,
        out_type=jax.ShapeDtypeStruct((E_G, F), jnp.float32),
        mesh=plsc.VectorSubcoreMesh(**_SC_MESH),
        compiler_params=pltpu.CompilerParams(use_tc_tiling_on_sc=False),
        scratch_types=[
            pltpu.VMEM((_H0_W,), jnp.int32),
            pltpu.VMEM((_H0_W,), jnp.int32),
            pltpu.VMEM((_H0_W, F), jnp.float32),
            pltpu.VMEM((_H0_W, F), jnp.float32),
            pltpu.SemaphoreType.DMA,
            pltpu.SemaphoreType.DMA,
        ],
    )(xa, xb, gsrc, gdst)


_DEG_CHUNK = 3200
_DEG_TEDGES = E_LG // NC // NS
_DEG_SLICE = E_G // NS


def _copy128(dst, src, off):
    for j in range(8):
        dst[pl.ds(j * 16, 16)] = src[pl.ds(off + j * 16, 16)]


def _deg_body(ldst_hbm, out_hbm, ibig, idxw, ones, zbuf, deg_sh, s1):
    cid = lax.axis_index("c")
    sid = lax.axis_index("s")

    def fill(j, _):
        ones[pl.ds(j * 16, 16)] = jnp.full((16,), 1.0, jnp.float32)
        return 0

    lax.fori_loop(0, 128 // 16, fill, 0)

    def zero(i, _):
        zbuf[pl.ds(i * 16, 16)] = jnp.zeros((16,), jnp.float32)
        return 0

    lax.fori_loop(0, _DEG_SLICE // 16, zero, 0, unroll=8)
    pltpu.sync_copy(zbuf, deg_sh.at[pl.ds(sid * _DEG_SLICE, _DEG_SLICE)])
    plsc.subcore_barrier()

    base = (cid * NS + sid) * _DEG_TEDGES

    def chunk(k, _):
        pltpu.sync_copy(ldst_hbm.at[pl.ds(base + k * _DEG_CHUNK, _DEG_CHUNK)],
                        ibig)

        def win(w, _):
            _copy128(idxw, ibig, w * 128)
            pltpu.sync_copy(ones, deg_sh.at[idxw], add=True)
            return 0

        lax.fori_loop(0, _DEG_CHUNK // 128, win, 0)
        return 0

    lax.fori_loop(0, _DEG_TEDGES // _DEG_CHUNK, chunk, 0)
    plsc.subcore_barrier()
    off = sid * _DEG_SLICE
    pltpu.sync_copy(deg_sh.at[pl.ds(off, _DEG_SLICE)], zbuf)
    pltpu.sync_copy(zbuf, out_hbm.at[cid, pl.ds(off, _DEG_SLICE)])


def _sc_deg(ldst):
    return pl.kernel(
        _deg_body,
        out_type=jax.ShapeDtypeStruct((NC, E_G), jnp.float32),
        mesh=plsc.VectorSubcoreMesh(**_SC_MESH),
        compiler_params=pltpu.CompilerParams(use_tc_tiling_on_sc=False),
        scratch_types=[
            pltpu.VMEM((_DEG_CHUNK,), jnp.int32),
            pltpu.VMEM((128,), jnp.int32),
            pltpu.VMEM((128,), jnp.float32),
            pltpu.VMEM((_DEG_SLICE,), jnp.float32),
            pltpu.VMEM_SHARED((E_G,), jnp.float32),
            pltpu.SemaphoreType.DMA,
        ],
    )(ldst)


_ACC_NB = 16
_ACC_R = E_G // _ACC_NB
_ACC_DUMP = _ACC_R
_ACC_S = 6400
_ACC_NWIN = _ACC_S // 128 + 1
_ACC_STRIP = E_LG // NS
_ACC_WSLICE = _ACC_R // NS


def _acc_body(p_hbm, lsrc_hbm, ldst_hbm, out_hbm,
              raw_s, raw_d, cs_flat, cd_flat,
              win_s0, win_d0, rows0, win_s1, win_d1, rows1,
              acc_sh, s1, g0, g1, t0, t1):
    cid = lax.axis_index("c")
    sid = lax.axis_index("s")
    iota = lax.broadcasted_iota(jnp.int32, (16,), 0)
    slots = ((win_s0, win_d0, rows0, g0, t0), (win_s1, win_d1, rows1, g1, t1))

    def window(d):
        _copy128(win_s0, cs_flat, d)
        _copy128(win_d0, cd_flat, d)
        pltpu.async_copy(p_hbm.at[win_s0], rows0, g0).wait()
        pltpu.sync_copy(rows0, acc_sh.at[win_d0], add=True)

    def pipelined_drain(ptr):
        def pair(q, _):
            for par in range(2):
                d = q * 2 + par - 2
                ws, wd, rw, gs, ts = slots[par]

                @pl.when((d >= 2) & ((d - 2) * 128 + 128 <= ptr))
                def _wait_scat():
                    pltpu.make_async_copy(rw, acc_sh.at[wd], ts).wait()

                @pl.when((d >= 0) & (d * 128 + 128 <= ptr))
                def _start_gather():
                    _copy128(ws, cs_flat, d * 128)
                    _copy128(wd, cd_flat, d * 128)
                    pltpu.async_copy(p_hbm.at[ws], rw, gs)

                wsp, wdp, rwp, gsp, tsp = slots[1 - par]

                @pl.when((d >= 1) & ((d - 1) * 128 + 128 <= ptr))
                def _start_scat():
                    pltpu.make_async_copy(p_hbm.at[wsp], rwp, gsp).wait()
                    pltpu.async_copy(rwp, acc_sh.at[wdp], tsp, add=True)

            return 0

        lax.fori_loop(0, (_ACC_NWIN + 6) // 2, pair, 0)

    for r_i in range(_ACC_NB // NC):
        rng = cid * (_ACC_NB // NC) + r_i
        lo = rng * _ACC_R
        off = sid * _ACC_WSLICE
        pltpu.sync_copy(p_hbm.at[pl.ds(lo + off, _ACC_WSLICE)],
                        acc_sh.at[pl.ds(off, _ACC_WSLICE)])
        plsc.subcore_barrier()

        def subchunk(k, ptr):
            pltpu.sync_copy(
                lsrc_hbm.at[pl.ds(sid * _ACC_STRIP + k * _ACC_S, _ACC_S)],
                raw_s)
            pltpu.sync_copy(
                ldst_hbm.at[pl.ds(sid * _ACC_STRIP + k * _ACC_S, _ACC_S)],
                raw_d)

            def scan(i, ptr):
                vs = raw_s[pl.ds(i * 16, 16)]
                vd = raw_d[pl.ds(i * 16, 16)]
                m = (vd >= lo) & (vd < lo + _ACC_R)
                csum = plsc.cumsum(m.astype(jnp.int32))
                pos = ptr + csum - 1
                plsc.store_scatter(cs_flat, [pos], vs, mask=m)
                plsc.store_scatter(cd_flat, [pos], vd - lo, mask=m)
                cnt = plsc.all_reduce_population_count(m)
                return ptr + cnt[0]

            ptr = lax.fori_loop(0, _ACC_S // 16, scan, ptr, unroll=4)

            pipelined_drain(ptr)
            done = (ptr // 128) * 128
            for j in range(8):
                sl = pl.ds(j * 16, 16)
                cs_flat[sl] = cs_flat[pl.ds(done + j * 16, 16)]
                cd_flat[sl] = cd_flat[pl.ds(done + j * 16, 16)]
            return ptr - done

        ptr = lax.fori_loop(0, _ACC_STRIP // _ACC_S, subchunk, 0)

        @pl.when(ptr > 0)
        def _flush():
            for j in range(8):
                lane = j * 16 + iota
                mf = lane >= ptr
                plsc.store_scatter(cs_flat, [lane],
                                   jnp.zeros((16,), jnp.int32), mask=mf)
                plsc.store_scatter(cd_flat, [lane],
                                   jnp.full((16,), _ACC_DUMP, jnp.int32),
                                   mask=mf)
            window(0)

        plsc.subcore_barrier()
        pltpu.sync_copy(acc_sh.at[pl.ds(off, _ACC_WSLICE)],
                        out_hbm.at[pl.ds(lo + off, _ACC_WSLICE)])


def _sc_acc(p, lsrc, ldst):
    return pl.kernel(
        _acc_body,
        out_type=jax.ShapeDtypeStruct((E_G, F), jnp.float32),
        mesh=plsc.VectorSubcoreMesh(**_SC_MESH),
        compiler_params=pltpu.CompilerParams(use_tc_tiling_on_sc=False,
                                             needs_layout_passes=False),
        scratch_types=[
            pltpu.VMEM((_ACC_S,), jnp.int32),
            pltpu.VMEM((_ACC_S,), jnp.int32),
            pltpu.VMEM((8192,), jnp.int32),
            pltpu.VMEM((8192,), jnp.int32),
            pltpu.VMEM((128,), jnp.int32),
            pltpu.VMEM((128,), jnp.int32),
            pltpu.VMEM((128, F), jnp.float32),
            pltpu.VMEM((128,), jnp.int32),
            pltpu.VMEM((128,), jnp.int32),
            pltpu.VMEM((128, F), jnp.float32),
            pltpu.VMEM_SHARED((_ACC_R + 8, F), jnp.float32),
            pltpu.SemaphoreType.DMA,
            pltpu.SemaphoreType.DMA,
            pltpu.SemaphoreType.DMA,
            pltpu.SemaphoreType.DMA,
            pltpu.SemaphoreType.DMA,
        ],
    )(p, lsrc, ldst)


def _mm_bias_body(a_ref, w_ref, b_ref, o_ref):
    o_ref[...] = jnp.dot(a_ref[...], w_ref[...],
                         preferred_element_type=jnp.float32) + b_ref[...]


def _mm_bias(a, w, b, blk):
    n, k = a.shape
    m = w.shape[1]
    return pl.pallas_call(
        _mm_bias_body,
        grid=(n // blk,),
        in_specs=[
            pl.BlockSpec((blk, k), lambda i: (i, 0)),
            pl.BlockSpec((k, m), lambda i: (0, 0)),
            pl.BlockSpec((1, m), lambda i: (0, 0)),
        ],
        out_specs=pl.BlockSpec((blk, m), lambda i: (i, 0)),
        out_shape=jax.ShapeDtypeStruct((n, m), jnp.float32),
    )(a, w, b)


def _fused_layer_body(a_ref, u_ref, v_ref, b_ref, w_ref, c_ref, o_ref):
    h = jax.nn.relu(u_ref[...] * a_ref[...] + b_ref[...])
    o_ref[...] = jnp.dot(v_ref[...] * h, w_ref[...],
                         preferred_element_type=jnp.float32) + c_ref[...]


def _fused_layer(a, u, v, b, w, c, blk):
    n, k = a.shape
    m = w.shape[1]
    return pl.pallas_call(
        _fused_layer_body,
        grid=(n // blk,),
        in_specs=[
            pl.BlockSpec((blk, k), lambda i: (i, 0)),
            pl.BlockSpec((blk, 1), lambda i: (i, 0)),
            pl.BlockSpec((blk, 1), lambda i: (i, 0)),
            pl.BlockSpec((1, k), lambda i: (0, 0)),
            pl.BlockSpec((k, m), lambda i: (0, 0)),
            pl.BlockSpec((1, m), lambda i: (0, 0)),
        ],
        out_specs=pl.BlockSpec((blk, m), lambda i: (i, 0)),
        out_shape=jax.ShapeDtypeStruct((n, m), jnp.float32),
    )(a, u, v, b, w, c)


def _dis_body(deg_ref, o_ref):
    o_ref[...] = jax.lax.rsqrt(deg_ref[0:1, :] + deg_ref[1:2, :] + 1.0)


def _dis_from_deg(degp, blk):
    n = degp.shape[1]
    return pl.pallas_call(
        _dis_body,
        out_shape=jax.ShapeDtypeStruct((1, n), jnp.float32),
    )(degp)


def kernel(x, g_edge_index, lg_edge_index, W1, b1, Wc1, bc1, Wc2, bc2, W2, b2):
    f32 = jnp.float32
    Wp = jnp.zeros((D_IN, 2 * F), f32)
    Wp = Wp.at[:, :HIDDEN].set(W1[:D_IN])
    Wp = Wp.at[:, F:F + HIDDEN].set(W1[D_IN:])
    bp = jnp.zeros((1, 2 * F), f32).at[0, F:F + HIDDEN].set(b1)
    Wc1p = jnp.zeros((F, F), f32).at[:HIDDEN, :HIDDEN].set(Wc1)
    bc1p = jnp.zeros((1, F), f32).at[0, :HIDDEN].set(bc1)
    Wc2p = jnp.zeros((F, F), f32).at[:HIDDEN, :HIDDEN].set(Wc2)
    bc2p = jnp.zeros((1, F), f32).at[0, :HIDDEN].set(bc2)
    W2p = jnp.zeros((F, 1), f32).at[:HIDDEN, :].set(W2)
    b2p = b2.reshape(1, 1)
    ones_n = jnp.ones((E_G, 1), f32)
    zeros_f = jnp.zeros((1, F), f32)
    zeros_1 = jnp.zeros((1, 1), f32)

    Zx = _mm_bias(x, Wp, bp, blk=2000)
    xa = Zx[:, :F]
    xb = Zx[:, F:]

    gsrc, gdst = g_edge_index[0], g_edge_index[1]
    lsrc, ldst = lg_edge_index[0], lg_edge_index[1]

    h0 = _sc_h0(xa, xb, gsrc, gdst)

    degp = _sc_deg(ldst)
    dis = _dis_from_deg(degp, blk=8000).reshape(E_G, 1)

    P1 = _fused_layer(h0, ones_n, dis, zeros_f, Wc1p, zeros_f, blk=4000)
    acc1 = _sc_acc(P1, lsrc, ldst)
    P2 = _fused_layer(acc1, dis, dis, bc1p, Wc2p, zeros_f, blk=4000)
    acc2 = _sc_acc(P2, lsrc, ldst)
    out = _fused_layer(acc2, dis, ones_n, bc2p, W2p, b2p, blk=4000)
    return out


# speedup vs baseline: 11.5407x; 1.1630x over previous
import functools

import jax
import jax.numpy as jnp
import numpy as np
from jax import lax
from jax.experimental import pallas as pl
from jax.experimental.pallas import tpu as pltpu
from jax.experimental.pallas import tpu_sc as plsc

N_NODES = 10000
D_IN = 128
E_G = 320000
E_LG = 5120000
HIDDEN = 52
F = 64
FP = 56

NC, NS = 2, 16
NW = NC * NS
_SC_MESH = dict(core_axis_name="c", subcore_axis_name="s")


def _wid():
    return lax.axis_index("s") * NC + lax.axis_index("c")


_H0_W = 128
_H0_PER = E_G // NW


def _h0_window(xa_hbm, xb_hbm, src_hbm, dst_hbm, out_hbm, isrc, idst, ra, rb,
               s1, s2, off, n):
    pltpu.sync_copy(src_hbm.at[pl.ds(off, n)], isrc.at[pl.ds(0, n)])
    pltpu.sync_copy(dst_hbm.at[pl.ds(off, n)], idst.at[pl.ds(0, n)])
    ca = pltpu.async_copy(xa_hbm.at[isrc.at[pl.ds(0, n)]], ra.at[pl.ds(0, n)], s1)
    cb = pltpu.async_copy(xb_hbm.at[idst.at[pl.ds(0, n)]], rb.at[pl.ds(0, n)], s2)
    ca.wait()
    cb.wait()

    def row(r, _):
        for j in range(F // 16):
            sl = pl.ds(j * 16, 16)
            ra[r, sl] = jnp.maximum(ra[r, sl] + rb[r, sl], 0.0)
        return 0

    lax.fori_loop(0, n, row, 0, unroll=4)
    pltpu.sync_copy(ra.at[pl.ds(0, n)], out_hbm.at[pl.ds(off, n)])


def _h0_body(xa_hbm, xb_hbm, src_hbm, dst_hbm, out_hbm, isrc, idst, ra, rb,
             s1, s2):
    base = _wid() * _H0_PER
    nwin = _H0_PER // _H0_W

    def win(w, _):
        _h0_window(xa_hbm, xb_hbm, src_hbm, dst_hbm, out_hbm, isrc, idst,
                   ra, rb, s1, s2, base + w * _H0_W, _H0_W)
        return 0

    lax.fori_loop(0, nwin, win, 0)
    tail = _H0_PER - nwin * _H0_W
    if tail:
        _h0_window(xa_hbm, xb_hbm, src_hbm, dst_hbm, out_hbm, isrc, idst,
                   ra, rb, s1, s2, base + nwin * _H0_W, tail)


def _sc_h0(xa, xb, gsrc, gdst):
    return pl.kernel(
        _h0_body,
        out_type=jax.ShapeDtypeStruct((E_G, F), jnp.float32),
        mesh=plsc.VectorSubcoreMesh(**_SC_MESH),
        compiler_params=pltpu.CompilerParams(use_tc_tiling_on_sc=False),
        scratch_types=[
            pltpu.VMEM((_H0_W,), jnp.int32),
            pltpu.VMEM((_H0_W,), jnp.int32),
            pltpu.VMEM((_H0_W, F), jnp.float32),
            pltpu.VMEM((_H0_W, F), jnp.float32),
            pltpu.SemaphoreType.DMA,
            pltpu.SemaphoreType.DMA,
        ],
    )(xa, xb, gsrc, gdst)


_DEG_CHUNK = 3200
_DEG_TEDGES = E_LG // NC // NS
_DEG_SLICE = E_G // NS


def _copy128(dst, src, off):
    for j in range(8):
        dst[pl.ds(j * 16, 16)] = src[pl.ds(off + j * 16, 16)]


def _copy64(dst, src, off):
    for j in range(4):
        dst[pl.ds(j * 16, 16)] = src[pl.ds(off + j * 16, 16)]


def _deg_body(ldst_hbm, out_hbm, ibig, idxw, ones, zbuf, deg_sh, s1):
    cid = lax.axis_index("c")
    sid = lax.axis_index("s")

    def fill(j, _):
        ones[pl.ds(j * 16, 16)] = jnp.full((16,), 1.0, jnp.float32)
        return 0

    lax.fori_loop(0, 128 // 16, fill, 0)

    def zero(i, _):
        zbuf[pl.ds(i * 16, 16)] = jnp.zeros((16,), jnp.float32)
        return 0

    lax.fori_loop(0, _DEG_SLICE // 16, zero, 0, unroll=8)
    pltpu.sync_copy(zbuf, deg_sh.at[pl.ds(sid * _DEG_SLICE, _DEG_SLICE)])
    plsc.subcore_barrier()

    base = (cid * NS + sid) * _DEG_TEDGES

    def chunk(k, _):
        pltpu.sync_copy(ldst_hbm.at[pl.ds(base + k * _DEG_CHUNK, _DEG_CHUNK)],
                        ibig)

        def win(w, _):
            _copy128(idxw, ibig, w * 128)
            pltpu.sync_copy(ones, deg_sh.at[idxw], add=True)
            return 0

        lax.fori_loop(0, _DEG_CHUNK // 128, win, 0)
        return 0

    lax.fori_loop(0, _DEG_TEDGES // _DEG_CHUNK, chunk, 0)
    plsc.subcore_barrier()
    off = sid * _DEG_SLICE
    pltpu.sync_copy(deg_sh.at[pl.ds(off, _DEG_SLICE)], zbuf)
    pltpu.sync_copy(zbuf, out_hbm.at[cid, pl.ds(off, _DEG_SLICE)])


def _sc_deg(ldst):
    return pl.kernel(
        _deg_body,
        out_type=jax.ShapeDtypeStruct((NC, E_G), jnp.float32),
        mesh=plsc.VectorSubcoreMesh(**_SC_MESH),
        compiler_params=pltpu.CompilerParams(use_tc_tiling_on_sc=False),
        scratch_types=[
            pltpu.VMEM((_DEG_CHUNK,), jnp.int32),
            pltpu.VMEM((128,), jnp.int32),
            pltpu.VMEM((128,), jnp.float32),
            pltpu.VMEM((_DEG_SLICE,), jnp.float32),
            pltpu.VMEM_SHARED((E_G,), jnp.float32),
            pltpu.SemaphoreType.DMA,
        ],
    )(ldst)


_ACC_NB = 10
_ACC_R = E_G // _ACC_NB
_ACC_DUMP = _ACC_R
_ACC_S = 2560
_ACC_W = 64
_ACC_FLAT = 2688
_ACC_NWIN = _ACC_S // _ACC_W + 1
_ACC_STRIP = E_LG // NS
_ACC_WSLICE = _ACC_R // NS


def _acc_body(p_hbm, lsrc_hbm, ldst_hbm, out_hbm,
              raw_s, raw_d, cs_flat, cd_flat,
              win_s0, win_d0, rows0, win_s1, win_d1, rows1,
              acc_sh, s1, g0, g1, t0, t1):
    cid = lax.axis_index("c")
    sid = lax.axis_index("s")
    iota = lax.broadcasted_iota(jnp.int32, (16,), 0)
    slots = ((win_s0, win_d0, rows0, g0, t0), (win_s1, win_d1, rows1, g1, t1))

    def window(d):
        _copy64(win_s0, cs_flat, d)
        _copy64(win_d0, cd_flat, d)
        pltpu.async_copy(p_hbm.at[win_s0], rows0, g0).wait()
        pltpu.sync_copy(rows0, acc_sh.at[win_d0], add=True)

    def pipelined_drain(ptr):
        def pair(q, _):
            for par in range(2):
                d = q * 2 + par - 2
                ws, wd, rw, gs, ts = slots[par]

                @pl.when((d >= 2) & ((d - 2) * _ACC_W + _ACC_W <= ptr))
                def _wait_scat():
                    pltpu.make_async_copy(rw, acc_sh.at[wd], ts).wait()

                @pl.when((d >= 0) & (d * _ACC_W + _ACC_W <= ptr))
                def _start_gather():
                    _copy64(ws, cs_flat, d * _ACC_W)
                    _copy64(wd, cd_flat, d * _ACC_W)
                    pltpu.async_copy(p_hbm.at[ws], rw, gs)

                wsp, wdp, rwp, gsp, tsp = slots[1 - par]

                @pl.when((d >= 1) & ((d - 1) * _ACC_W + _ACC_W <= ptr))
                def _start_scat():
                    pltpu.make_async_copy(p_hbm.at[wsp], rwp, gsp).wait()
                    pltpu.async_copy(rwp, acc_sh.at[wdp], tsp, add=True)

            return 0

        lax.fori_loop(0, (_ACC_NWIN + 6) // 2, pair, 0)

    for r_i in range(_ACC_NB // NC):
        rng = cid * (_ACC_NB // NC) + r_i
        lo = rng * _ACC_R
        off = sid * _ACC_WSLICE
        pltpu.sync_copy(p_hbm.at[pl.ds(lo + off, _ACC_WSLICE)],
                        acc_sh.at[pl.ds(off, _ACC_WSLICE)])
        plsc.subcore_barrier()

        def subchunk(k, ptr):
            pltpu.sync_copy(
                lsrc_hbm.at[pl.ds(sid * _ACC_STRIP + k * _ACC_S, _ACC_S)],
                raw_s)
            pltpu.sync_copy(
                ldst_hbm.at[pl.ds(sid * _ACC_STRIP + k * _ACC_S, _ACC_S)],
                raw_d)

            def scan(i, ptr):
                vs = raw_s[pl.ds(i * 16, 16)]
                vd = raw_d[pl.ds(i * 16, 16)]
                m = (vd >= lo) & (vd < lo + _ACC_R)
                csum = plsc.cumsum(m.astype(jnp.int32))
                pos = ptr + csum - 1
                plsc.store_scatter(cs_flat, [pos], vs, mask=m)
                plsc.store_scatter(cd_flat, [pos], vd - lo, mask=m)
                cnt = plsc.all_reduce_population_count(m)
                return ptr + cnt[0]

            ptr = lax.fori_loop(0, _ACC_S // 16, scan, ptr, unroll=4)

            pipelined_drain(ptr)
            done = (ptr // _ACC_W) * _ACC_W
            for j in range(4):
                sl = pl.ds(j * 16, 16)
                cs_flat[sl] = cs_flat[pl.ds(done + j * 16, 16)]
                cd_flat[sl] = cd_flat[pl.ds(done + j * 16, 16)]
            return ptr - done

        ptr = lax.fori_loop(0, _ACC_STRIP // _ACC_S, subchunk, 0)

        @pl.when(ptr > 0)
        def _flush():
            for j in range(4):
                lane = j * 16 + iota
                mf = lane >= ptr
                plsc.store_scatter(cs_flat, [lane],
                                   jnp.zeros((16,), jnp.int32), mask=mf)
                plsc.store_scatter(cd_flat, [lane],
                                   jnp.full((16,), _ACC_DUMP, jnp.int32),
                                   mask=mf)
            window(0)

        plsc.subcore_barrier()
        pltpu.sync_copy(acc_sh.at[pl.ds(off, _ACC_WSLICE)],
                        out_hbm.at[pl.ds(lo + off, _ACC_WSLICE)])


def _sc_acc(p, lsrc, ldst):
    return pl.kernel(
        _acc_body,
        out_type=jax.ShapeDtypeStruct((E_G, FP), jnp.float32),
        mesh=plsc.VectorSubcoreMesh(**_SC_MESH),
        compiler_params=pltpu.CompilerParams(use_tc_tiling_on_sc=False,
                                             needs_layout_passes=False),
        scratch_types=[
            pltpu.VMEM((_ACC_S,), jnp.int32),
            pltpu.VMEM((_ACC_S,), jnp.int32),
            pltpu.VMEM((_ACC_FLAT,), jnp.int32),
            pltpu.VMEM((_ACC_FLAT,), jnp.int32),
            pltpu.VMEM((_ACC_W,), jnp.int32),
            pltpu.VMEM((_ACC_W,), jnp.int32),
            pltpu.VMEM((_ACC_W, FP), jnp.float32),
            pltpu.VMEM((_ACC_W,), jnp.int32),
            pltpu.VMEM((_ACC_W,), jnp.int32),
            pltpu.VMEM((_ACC_W, FP), jnp.float32),
            pltpu.VMEM_SHARED((_ACC_R + 8, FP), jnp.float32),
            pltpu.SemaphoreType.DMA,
            pltpu.SemaphoreType.DMA,
            pltpu.SemaphoreType.DMA,
            pltpu.SemaphoreType.DMA,
            pltpu.SemaphoreType.DMA,
        ],
    )(p, lsrc, ldst)


def _mm_bias_body(a_ref, w_ref, b_ref, o_ref):
    o_ref[...] = jnp.dot(a_ref[...], w_ref[...],
                         preferred_element_type=jnp.float32) + b_ref[...]


def _mm_bias(a, w, b, blk):
    n, k = a.shape
    m = w.shape[1]
    return pl.pallas_call(
        _mm_bias_body,
        grid=(n // blk,),
        in_specs=[
            pl.BlockSpec((blk, k), lambda i: (i, 0)),
            pl.BlockSpec((k, m), lambda i: (0, 0)),
            pl.BlockSpec((1, m), lambda i: (0, 0)),
        ],
        out_specs=pl.BlockSpec((blk, m), lambda i: (i, 0)),
        out_shape=jax.ShapeDtypeStruct((n, m), jnp.float32),
    )(a, w, b)


def _fused_layer_body(a_ref, u_ref, v_ref, b_ref, w_ref, c_ref, o_ref):
    h = jax.nn.relu(u_ref[...] * a_ref[...] + b_ref[...])
    o_ref[...] = jnp.dot(v_ref[...] * h, w_ref[...],
                         preferred_element_type=jnp.float32) + c_ref[...]


def _fused_layer(a, u, v, b, w, c, blk):
    n, k = a.shape
    m = w.shape[1]
    return pl.pallas_call(
        _fused_layer_body,
        grid=(n // blk,),
        in_specs=[
            pl.BlockSpec((blk, k), lambda i: (i, 0)),
            pl.BlockSpec((blk, 1), lambda i: (i, 0)),
            pl.BlockSpec((blk, 1), lambda i: (i, 0)),
            pl.BlockSpec((1, k), lambda i: (0, 0)),
            pl.BlockSpec((k, m), lambda i: (0, 0)),
            pl.BlockSpec((1, m), lambda i: (0, 0)),
        ],
        out_specs=pl.BlockSpec((blk, m), lambda i: (i, 0)),
        out_shape=jax.ShapeDtypeStruct((n, m), jnp.float32),
    )(a, u, v, b, w, c)


def _dis_body(deg_ref, o_ref):
    o_ref[...] = jax.lax.rsqrt(deg_ref[0:1, :] + deg_ref[1:2, :] + 1.0)


def _dis_from_deg(degp, blk):
    n = degp.shape[1]
    return pl.pallas_call(
        _dis_body,
        out_shape=jax.ShapeDtypeStruct((1, n), jnp.float32),
    )(degp)


def kernel(x, g_edge_index, lg_edge_index, W1, b1, Wc1, bc1, Wc2, bc2, W2, b2):
    f32 = jnp.float32
    Wp = jnp.zeros((D_IN, 2 * F), f32)
    Wp = Wp.at[:, :HIDDEN].set(W1[:D_IN])
    Wp = Wp.at[:, F:F + HIDDEN].set(W1[D_IN:])
    bp = jnp.zeros((1, 2 * F), f32).at[0, F:F + HIDDEN].set(b1)
    Wc1p = jnp.zeros((F, FP), f32).at[:HIDDEN, :HIDDEN].set(Wc1)
    bc1p = jnp.zeros((1, FP), f32).at[0, :HIDDEN].set(bc1)
    Wc2p = jnp.zeros((FP, FP), f32).at[:HIDDEN, :HIDDEN].set(Wc2)
    bc2p = jnp.zeros((1, FP), f32).at[0, :HIDDEN].set(bc2)
    W2p = jnp.zeros((FP, 1), f32).at[:HIDDEN, :].set(W2)
    b2p = b2.reshape(1, 1)
    ones_n = jnp.ones((E_G, 1), f32)
    zeros_f = jnp.zeros((1, F), f32)
    zeros_fp = jnp.zeros((1, FP), f32)

    Zx = _mm_bias(x, Wp, bp, blk=2000)
    xa = Zx[:, :F]
    xb = Zx[:, F:]

    gsrc, gdst = g_edge_index[0], g_edge_index[1]
    lsrc, ldst = lg_edge_index[0], lg_edge_index[1]

    h0 = _sc_h0(xa, xb, gsrc, gdst)

    degp = _sc_deg(ldst)
    dis = _dis_from_deg(degp, blk=8000).reshape(E_G, 1)

    P1 = _fused_layer(h0, ones_n, dis, zeros_f, Wc1p, zeros_fp, blk=4000)
    acc1 = _sc_acc(P1, lsrc, ldst)
    P2 = _fused_layer(acc1, dis, dis, bc1p, Wc2p, zeros_fp, blk=4000)
    acc2 = _sc_acc(P2, lsrc, ldst)
    out = _fused_layer(acc2, dis, ones_n, bc2p, W2p, b2p, blk=4000)
    return out


# speedup vs baseline: 11.6147x; 1.0064x over previous
import functools

import jax
import jax.numpy as jnp
import numpy as np
from jax import lax
from jax.experimental import pallas as pl
from jax.experimental.pallas import tpu as pltpu
from jax.experimental.pallas import tpu_sc as plsc

N_NODES = 10000
D_IN = 128
E_G = 320000
E_LG = 5120000
HIDDEN = 52
F = 64
FP = 56

NC, NS = 2, 16
NW = NC * NS
_SC_MESH = dict(core_axis_name="c", subcore_axis_name="s")


def _wid():
    return lax.axis_index("s") * NC + lax.axis_index("c")


_H0_W = 128
_H0_PER = E_G // NW


def _h0_window(xa_hbm, xb_hbm, src_hbm, dst_hbm, out_hbm, isrc, idst, ra, rb,
               s1, s2, off, n):
    pltpu.sync_copy(src_hbm.at[pl.ds(off, n)], isrc.at[pl.ds(0, n)])
    pltpu.sync_copy(dst_hbm.at[pl.ds(off, n)], idst.at[pl.ds(0, n)])
    ca = pltpu.async_copy(xa_hbm.at[isrc.at[pl.ds(0, n)]], ra.at[pl.ds(0, n)], s1)
    cb = pltpu.async_copy(xb_hbm.at[idst.at[pl.ds(0, n)]], rb.at[pl.ds(0, n)], s2)
    ca.wait()
    cb.wait()

    def row(r, _):
        for j in range(F // 16):
            sl = pl.ds(j * 16, 16)
            ra[r, sl] = jnp.maximum(ra[r, sl] + rb[r, sl], 0.0)
        return 0

    lax.fori_loop(0, n, row, 0, unroll=4)
    pltpu.sync_copy(ra.at[pl.ds(0, n)], out_hbm.at[pl.ds(off, n)])


def _h0_body(xa_hbm, xb_hbm, src_hbm, dst_hbm, out_hbm, isrc, idst, ra, rb,
             s1, s2):
    base = _wid() * _H0_PER
    nwin = _H0_PER // _H0_W

    def win(w, _):
        _h0_window(xa_hbm, xb_hbm, src_hbm, dst_hbm, out_hbm, isrc, idst,
                   ra, rb, s1, s2, base + w * _H0_W, _H0_W)
        return 0

    lax.fori_loop(0, nwin, win, 0)
    tail = _H0_PER - nwin * _H0_W
    if tail:
        _h0_window(xa_hbm, xb_hbm, src_hbm, dst_hbm, out_hbm, isrc, idst,
                   ra, rb, s1, s2, base + nwin * _H0_W, tail)


def _sc_h0(xa, xb, gsrc, gdst):
    return pl.kernel(
        _h0_body,
        out_type=jax.ShapeDtypeStruct((E_G, F), jnp.float32),
        mesh=plsc.VectorSubcoreMesh(**_SC_MESH),
        compiler_params=pltpu.CompilerParams(use_tc_tiling_on_sc=False),
        scratch_types=[
            pltpu.VMEM((_H0_W,), jnp.int32),
            pltpu.VMEM((_H0_W,), jnp.int32),
            pltpu.VMEM((_H0_W, F), jnp.float32),
            pltpu.VMEM((_H0_W, F), jnp.float32),
            pltpu.SemaphoreType.DMA,
            pltpu.SemaphoreType.DMA,
        ],
    )(xa, xb, gsrc, gdst)


_DEG_CHUNK = 3200
_DEG_TEDGES = E_LG // NC // NS
_DEG_SLICE = E_G // NS


def _copy128(dst, src, off):
    for j in range(8):
        dst[pl.ds(j * 16, 16)] = src[pl.ds(off + j * 16, 16)]


def _copy64(dst, src, off):
    for j in range(4):
        dst[pl.ds(j * 16, 16)] = src[pl.ds(off + j * 16, 16)]


def _deg_body(ldst_hbm, out_hbm, ibig, idxw, ones, zbuf, deg_sh, s1):
    cid = lax.axis_index("c")
    sid = lax.axis_index("s")

    def fill(j, _):
        ones[pl.ds(j * 16, 16)] = jnp.full((16,), 1.0, jnp.float32)
        return 0

    lax.fori_loop(0, 128 // 16, fill, 0)

    def zero(i, _):
        zbuf[pl.ds(i * 16, 16)] = jnp.zeros((16,), jnp.float32)
        return 0

    lax.fori_loop(0, _DEG_SLICE // 16, zero, 0, unroll=8)
    pltpu.sync_copy(zbuf, deg_sh.at[pl.ds(sid * _DEG_SLICE, _DEG_SLICE)])
    plsc.subcore_barrier()

    base = (cid * NS + sid) * _DEG_TEDGES

    def chunk(k, _):
        pltpu.sync_copy(ldst_hbm.at[pl.ds(base + k * _DEG_CHUNK, _DEG_CHUNK)],
                        ibig)

        def win(w, _):
            _copy128(idxw, ibig, w * 128)
            pltpu.sync_copy(ones, deg_sh.at[idxw], add=True)
            return 0

        lax.fori_loop(0, _DEG_CHUNK // 128, win, 0)
        return 0

    lax.fori_loop(0, _DEG_TEDGES // _DEG_CHUNK, chunk, 0)
    plsc.subcore_barrier()
    off = sid * _DEG_SLICE
    pltpu.sync_copy(deg_sh.at[pl.ds(off, _DEG_SLICE)], zbuf)
    pltpu.sync_copy(zbuf, out_hbm.at[cid, pl.ds(off, _DEG_SLICE)])


def _sc_deg(ldst):
    return pl.kernel(
        _deg_body,
        out_type=jax.ShapeDtypeStruct((NC, E_G), jnp.float32),
        mesh=plsc.VectorSubcoreMesh(**_SC_MESH),
        compiler_params=pltpu.CompilerParams(use_tc_tiling_on_sc=False),
        scratch_types=[
            pltpu.VMEM((_DEG_CHUNK,), jnp.int32),
            pltpu.VMEM((128,), jnp.int32),
            pltpu.VMEM((128,), jnp.float32),
            pltpu.VMEM((_DEG_SLICE,), jnp.float32),
            pltpu.VMEM_SHARED((E_G,), jnp.float32),
            pltpu.SemaphoreType.DMA,
        ],
    )(ldst)


_ACC_NB = 10
_ACC_R = E_G // _ACC_NB
_ACC_DUMP = _ACC_R
_ACC_S = 2560
_ACC_W = 64
_ACC_FLAT = 2688
_ACC_NWIN = _ACC_S // _ACC_W + 1
_ACC_STRIP = E_LG // NS
_ACC_WSLICE = _ACC_R // NS


def _acc_body(p_hbm, lsrc_hbm, ldst_hbm, out_hbm,
              raw_s, raw_d, cs_flat, cd_flat,
              win_s0, win_d0, rows0, win_s1, win_d1, rows1,
              acc_sh, s1, g0, g1, t0, t1):
    cid = lax.axis_index("c")
    sid = lax.axis_index("s")
    iota = lax.broadcasted_iota(jnp.int32, (16,), 0)
    slots = ((win_s0, win_d0, rows0, g0, t0), (win_s1, win_d1, rows1, g1, t1))

    def window(d):
        _copy64(win_s0, cs_flat, d)
        _copy64(win_d0, cd_flat, d)
        pltpu.async_copy(p_hbm.at[win_s0], rows0, g0).wait()
        pltpu.sync_copy(rows0, acc_sh.at[win_d0], add=True)

    def pipelined_drain(ptr):
        def pair(q, _):
            for par in range(2):
                d = q * 2 + par - 2
                ws, wd, rw, gs, ts = slots[par]

                @pl.when((d >= 2) & ((d - 2) * _ACC_W + _ACC_W <= ptr))
                def _wait_scat():
                    pltpu.make_async_copy(rw, acc_sh.at[wd], ts).wait()

                @pl.when((d >= 0) & (d * _ACC_W + _ACC_W <= ptr))
                def _start_gather():
                    _copy64(ws, cs_flat, d * _ACC_W)
                    _copy64(wd, cd_flat, d * _ACC_W)
                    pltpu.async_copy(p_hbm.at[ws], rw, gs)

                wsp, wdp, rwp, gsp, tsp = slots[1 - par]

                @pl.when((d >= 1) & ((d - 1) * _ACC_W + _ACC_W <= ptr))
                def _start_scat():
                    pltpu.make_async_copy(p_hbm.at[wsp], rwp, gsp).wait()
                    pltpu.async_copy(rwp, acc_sh.at[wdp], tsp, add=True)

            return 0

        lax.fori_loop(0, (_ACC_NWIN + 6) // 2, pair, 0)

    for r_i in range(_ACC_NB // NC):
        rng = cid * (_ACC_NB // NC) + r_i
        lo = rng * _ACC_R
        off = sid * _ACC_WSLICE
        pltpu.sync_copy(p_hbm.at[pl.ds(lo + off, _ACC_WSLICE)],
                        acc_sh.at[pl.ds(off, _ACC_WSLICE)])
        plsc.subcore_barrier()

        def subchunk(k, ptr):
            pltpu.sync_copy(
                lsrc_hbm.at[pl.ds(sid * _ACC_STRIP + k * _ACC_S, _ACC_S)],
                raw_s)
            pltpu.sync_copy(
                ldst_hbm.at[pl.ds(sid * _ACC_STRIP + k * _ACC_S, _ACC_S)],
                raw_d)

            def scan(i, ptr):
                vs = raw_s[pl.ds(i * 16, 16)]
                vd = raw_d[pl.ds(i * 16, 16)]
                m = (vd >= lo) & (vd < lo + _ACC_R)
                csum = plsc.cumsum(m.astype(jnp.int32))
                pos = ptr + csum - 1
                plsc.store_scatter(cs_flat, [pos], vs, mask=m)
                plsc.store_scatter(cd_flat, [pos], vd - lo, mask=m)
                cnt = plsc.all_reduce_population_count(m)
                return ptr + cnt[0]

            ptr = lax.fori_loop(0, _ACC_S // 16, scan, ptr, unroll=8)

            pipelined_drain(ptr)
            done = (ptr // _ACC_W) * _ACC_W
            for j in range(4):
                sl = pl.ds(j * 16, 16)
                cs_flat[sl] = cs_flat[pl.ds(done + j * 16, 16)]
                cd_flat[sl] = cd_flat[pl.ds(done + j * 16, 16)]
            return ptr - done

        ptr = lax.fori_loop(0, _ACC_STRIP // _ACC_S, subchunk, 0)

        @pl.when(ptr > 0)
        def _flush():
            for j in range(4):
                lane = j * 16 + iota
                mf = lane >= ptr
                plsc.store_scatter(cs_flat, [lane],
                                   jnp.zeros((16,), jnp.int32), mask=mf)
                plsc.store_scatter(cd_flat, [lane],
                                   jnp.full((16,), _ACC_DUMP, jnp.int32),
                                   mask=mf)
            window(0)

        plsc.subcore_barrier()
        pltpu.sync_copy(acc_sh.at[pl.ds(off, _ACC_WSLICE)],
                        out_hbm.at[pl.ds(lo + off, _ACC_WSLICE)])


def _sc_acc(p, lsrc, ldst):
    return pl.kernel(
        _acc_body,
        out_type=jax.ShapeDtypeStruct((E_G, FP), jnp.float32),
        mesh=plsc.VectorSubcoreMesh(**_SC_MESH),
        compiler_params=pltpu.CompilerParams(use_tc_tiling_on_sc=False,
                                             needs_layout_passes=False),
        scratch_types=[
            pltpu.VMEM((_ACC_S,), jnp.int32),
            pltpu.VMEM((_ACC_S,), jnp.int32),
            pltpu.VMEM((_ACC_FLAT,), jnp.int32),
            pltpu.VMEM((_ACC_FLAT,), jnp.int32),
            pltpu.VMEM((_ACC_W,), jnp.int32),
            pltpu.VMEM((_ACC_W,), jnp.int32),
            pltpu.VMEM((_ACC_W, FP), jnp.float32),
            pltpu.VMEM((_ACC_W,), jnp.int32),
            pltpu.VMEM((_ACC_W,), jnp.int32),
            pltpu.VMEM((_ACC_W, FP), jnp.float32),
            pltpu.VMEM_SHARED((_ACC_R + 8, FP), jnp.float32),
            pltpu.SemaphoreType.DMA,
            pltpu.SemaphoreType.DMA,
            pltpu.SemaphoreType.DMA,
            pltpu.SemaphoreType.DMA,
            pltpu.SemaphoreType.DMA,
        ],
    )(p, lsrc, ldst)


def _mm_bias_body(a_ref, w_ref, b_ref, o_ref):
    o_ref[...] = jnp.dot(a_ref[...], w_ref[...],
                         preferred_element_type=jnp.float32) + b_ref[...]


def _mm_bias(a, w, b, blk):
    n, k = a.shape
    m = w.shape[1]
    return pl.pallas_call(
        _mm_bias_body,
        grid=(n // blk,),
        in_specs=[
            pl.BlockSpec((blk, k), lambda i: (i, 0)),
            pl.BlockSpec((k, m), lambda i: (0, 0)),
            pl.BlockSpec((1, m), lambda i: (0, 0)),
        ],
        out_specs=pl.BlockSpec((blk, m), lambda i: (i, 0)),
        out_shape=jax.ShapeDtypeStruct((n, m), jnp.float32),
    )(a, w, b)


def _fused_layer_body(a_ref, u_ref, v_ref, b_ref, w_ref, c_ref, o_ref):
    h = jax.nn.relu(u_ref[...] * a_ref[...] + b_ref[...])
    o_ref[...] = jnp.dot(v_ref[...] * h, w_ref[...],
                         preferred_element_type=jnp.float32) + c_ref[...]


def _fused_layer(a, u, v, b, w, c, blk):
    n, k = a.shape
    m = w.shape[1]
    return pl.pallas_call(
        _fused_layer_body,
        grid=(n // blk,),
        in_specs=[
            pl.BlockSpec((blk, k), lambda i: (i, 0)),
            pl.BlockSpec((blk, 1), lambda i: (i, 0)),
            pl.BlockSpec((blk, 1), lambda i: (i, 0)),
            pl.BlockSpec((1, k), lambda i: (0, 0)),
            pl.BlockSpec((k, m), lambda i: (0, 0)),
            pl.BlockSpec((1, m), lambda i: (0, 0)),
        ],
        out_specs=pl.BlockSpec((blk, m), lambda i: (i, 0)),
        out_shape=jax.ShapeDtypeStruct((n, m), jnp.float32),
    )(a, u, v, b, w, c)


def _dis_body(deg_ref, o_ref):
    o_ref[...] = jax.lax.rsqrt(deg_ref[0:1, :] + deg_ref[1:2, :] + 1.0)


def _dis_from_deg(degp, blk):
    n = degp.shape[1]
    return pl.pallas_call(
        _dis_body,
        out_shape=jax.ShapeDtypeStruct((1, n), jnp.float32),
    )(degp)


def kernel(x, g_edge_index, lg_edge_index, W1, b1, Wc1, bc1, Wc2, bc2, W2, b2):
    f32 = jnp.float32
    Wp = jnp.zeros((D_IN, 2 * F), f32)
    Wp = Wp.at[:, :HIDDEN].set(W1[:D_IN])
    Wp = Wp.at[:, F:F + HIDDEN].set(W1[D_IN:])
    bp = jnp.zeros((1, 2 * F), f32).at[0, F:F + HIDDEN].set(b1)
    Wc1p = jnp.zeros((F, FP), f32).at[:HIDDEN, :HIDDEN].set(Wc1)
    bc1p = jnp.zeros((1, FP), f32).at[0, :HIDDEN].set(bc1)
    Wc2p = jnp.zeros((FP, FP), f32).at[:HIDDEN, :HIDDEN].set(Wc2)
    bc2p = jnp.zeros((1, FP), f32).at[0, :HIDDEN].set(bc2)
    W2p = jnp.zeros((FP, 1), f32).at[:HIDDEN, :].set(W2)
    b2p = b2.reshape(1, 1)
    ones_n = jnp.ones((E_G, 1), f32)
    zeros_f = jnp.zeros((1, F), f32)
    zeros_fp = jnp.zeros((1, FP), f32)

    Zx = _mm_bias(x, Wp, bp, blk=2000)
    xa = Zx[:, :F]
    xb = Zx[:, F:]

    gsrc, gdst = g_edge_index[0], g_edge_index[1]
    lsrc, ldst = lg_edge_index[0], lg_edge_index[1]

    h0 = _sc_h0(xa, xb, gsrc, gdst)

    degp = _sc_deg(ldst)
    dis = _dis_from_deg(degp, blk=8000).reshape(E_G, 1)

    P1 = _fused_layer(h0, ones_n, dis, zeros_f, Wc1p, zeros_fp, blk=4000)
    acc1 = _sc_acc(P1, lsrc, ldst)
    P2 = _fused_layer(acc1, dis, dis, bc1p, Wc2p, zeros_fp, blk=4000)
    acc2 = _sc_acc(P2, lsrc, ldst)
    out = _fused_layer(acc2, dis, ones_n, bc2p, W2p, b2p, blk=4000)
    return out
